# baseline (speedup 1.0000x reference)
import functools

import jax
import jax.numpy as jnp
from jax import lax
from jax.experimental import pallas as pl
from jax.experimental.pallas import tpu as pltpu
from jax.experimental.pallas import tpu_sc as plsc

NC = 2
NS = 16
NW = NC * NS

B, T, H, D = 4, 4096, 8, 64
TOTAL = B * T * H
V = 800000
NPW = TOTAL // NW
C = 128
NCHUNK = NPW // C
NB = 4


def _sc_gather(ids2d, offs16, table):
    mesh = plsc.VectorSubcoreMesh(core_axis_name="c", subcore_axis_name="s")

    @functools.partial(
        pl.kernel,
        out_type=jax.ShapeDtypeStruct((B, T, H, D), jnp.float32),
        mesh=mesh,
        compiler_params=pltpu.CompilerParams(
            use_tc_tiling_on_sc=True, needs_layout_passes=False
        ),
        scratch_types=[
            pltpu.VMEM((NCHUNK, C), jnp.int32),
            pltpu.VMEM((16,), jnp.int32),
            pltpu.VMEM((NB, C // H, H, D), jnp.float32),
            pltpu.SemaphoreType.DMA((NB,)),
            pltpu.SemaphoreType.DMA((NB,)),
        ],
    )
    def k(ids_hbm, offs_hbm, table_hbm, out_hbm, idx_v, offs_v, rows_v, gsem, osem):
        wid = lax.axis_index("s") * NC + lax.axis_index("c")
        bb = wid // (NW // B)
        tbase = (wid % (NW // B)) * (NPW // H)
        pltpu.sync_copy(ids_hbm.at[pl.ds(wid * NCHUNK, NCHUNK)], idx_v)
        pltpu.sync_copy(offs_hbm, offs_v)
        ov = offs_v[...]

        @pl.loop(0, NCHUNK)
        def _(j):
            for g in range(C // 16):
                sl = (j, pl.ds(g * 16, 16))
                idx_v[sl] = idx_v[sl] + ov

        def g_start(ch, b):
            @pl.loop(0, C // 16)
            def _(g):
                vec = idx_v[ch, pl.ds(g * 16, 16)]
                for lane in range(16):
                    pltpu.async_copy(
                        table_hbm.at[vec[lane]],
                        rows_v.at[b, (g * 16 + lane) // H, (g * 16 + lane) % H],
                        gsem.at[b],
                    )

        def g_wait(ch, b):
            @pl.loop(0, C, unroll=8)
            def _(i):
                pltpu.make_async_copy(
                    table_hbm.at[0], rows_v.at[b, 0, 0], gsem.at[b]
                ).wait()

        def o_start(ch, b):
            pltpu.async_copy(
                rows_v.at[b],
                out_hbm.at[bb, pl.ds(tbase + ch * (C // H), C // H)],
                osem.at[b],
            )

        def o_wait(ch, b):
            pltpu.make_async_copy(
                rows_v.at[b],
                out_hbm.at[bb, pl.ds(tbase + ch * (C // H), C // H)],
                osem.at[b],
            ).wait()

        for b in range(NB):
            g_start(b, b)

        @pl.loop(0, NCHUNK - NB, step=NB)
        def _(j):
            for b in range(NB):
                ch = j + b
                g_wait(ch, b)
                o_start(ch, b)
                o_wait(ch, b)
                g_start(ch + NB, b)

        for b in range(NB):
            ch = NCHUNK - NB + b
            g_wait(ch, b)
            o_start(ch, b)
        for b in range(NB):
            o_wait(NCHUNK - NB + b, b)

    return k


def kernel(hash_ids, table, offsets):
    ids2d = hash_ids.reshape(TOTAL // C, C)
    offs16 = jnp.concatenate([offsets, offsets]).astype(jnp.int32)
    table = lax.optimization_barrier(table)
    return _sc_gather(ids2d, offs16, table)(ids2d, offs16, table)


# speedup vs baseline: 1.0334x; 1.0334x over previous
import functools

import jax
import jax.numpy as jnp
from jax import lax
from jax.experimental import pallas as pl
from jax.experimental.pallas import tpu as pltpu
from jax.experimental.pallas import tpu_sc as plsc

NC = 2
NS = 16
NW = NC * NS

B, T, H, D = 4, 4096, 8, 64
TOTAL = B * T * H
V = 800000
NPW = TOTAL // NW
C = 128
NCHUNK = NPW // C
NB = 4


def _sc_gather(ids2d, offs16, table):
    mesh = plsc.VectorSubcoreMesh(core_axis_name="c", subcore_axis_name="s")

    @functools.partial(
        pl.kernel,
        out_type=jax.ShapeDtypeStruct((TOTAL, D), jnp.float32),
        mesh=mesh,
        compiler_params=pltpu.CompilerParams(
            use_tc_tiling_on_sc=True, needs_layout_passes=False
        ),
        scratch_types=[
            pltpu.VMEM((NCHUNK, C), jnp.int32),
            pltpu.VMEM((16,), jnp.int32),
            pltpu.VMEM((NB, C, D), jnp.float32),
            pltpu.SemaphoreType.DMA((NB,)),
            pltpu.SemaphoreType.DMA((NB,)),
        ],
    )
    def k(ids_hbm, offs_hbm, table_hbm, out_hbm, idx_v, offs_v, rows_v, gsem, osem):
        wid = lax.axis_index("s") * NC + lax.axis_index("c")
        rowbase = wid * NPW
        pltpu.sync_copy(ids_hbm.at[pl.ds(wid * NCHUNK, NCHUNK)], idx_v)
        pltpu.sync_copy(offs_hbm, offs_v)
        ov = offs_v[...]

        @pl.loop(0, NCHUNK)
        def _(j):
            for g in range(C // 16):
                sl = (j, pl.ds(g * 16, 16))
                idx_v[sl] = idx_v[sl] + ov

        def g_start(ch, b):
            @pl.loop(0, C // 16)
            def _(g):
                vec = idx_v[ch, pl.ds(g * 16, 16)]
                for lane in range(16):
                    pltpu.async_copy(
                        table_hbm.at[vec[lane]],
                        rows_v.at[b, g * 16 + lane],
                        gsem.at[b],
                    )

        def g_wait(ch, b):
            @pl.loop(0, C, unroll=8)
            def _(i):
                pltpu.make_async_copy(
                    table_hbm.at[0], rows_v.at[b, i], gsem.at[b]
                ).wait()

        def o_start(ch, b):
            pltpu.async_copy(
                rows_v.at[b], out_hbm.at[pl.ds(rowbase + ch * C, C)], osem.at[b]
            )

        def o_wait(ch, b):
            pltpu.make_async_copy(
                rows_v.at[b], out_hbm.at[pl.ds(rowbase + ch * C, C)], osem.at[b]
            ).wait()

        for b in range(NB):
            g_start(b, b)

        @pl.loop(0, NCHUNK - NB, step=NB)
        def _(j):
            for b in range(NB):
                ch = j + b
                g_wait(ch, b)
                o_start(ch, b)
                o_wait(ch, b)
                g_start(ch + NB, b)

        for b in range(NB):
            ch = NCHUNK - NB + b
            g_wait(ch, b)
            o_start(ch, b)
        for b in range(NB):
            o_wait(NCHUNK - NB + b, b)

    return k


def kernel(hash_ids, table, offsets):
    ids2d = hash_ids.reshape(TOTAL // C, C)
    offs16 = jnp.concatenate([offsets, offsets]).astype(jnp.int32)
    out = _sc_gather(ids2d, offs16, table)(ids2d, offs16, table)
    return out.reshape(B, T, H, D)


# speedup vs baseline: 1.5588x; 1.5084x over previous
import functools

import jax
import jax.numpy as jnp
from jax import lax
from jax.experimental import pallas as pl
from jax.experimental.pallas import tpu as pltpu
from jax.experimental.pallas import tpu_sc as plsc

NC = 2
NS = 16
NW = NC * NS

B, T, H, D = 4, 4096, 8, 64
TOTAL = B * T * H
V = 800000
NPW = TOTAL // NW
C = 128
NCHUNK = NPW // C
NB = 4


def _sc_gather(ids2d, offs16, table3):
    mesh = plsc.VectorSubcoreMesh(core_axis_name="c", subcore_axis_name="s")

    @functools.partial(
        pl.kernel,
        out_type=jax.ShapeDtypeStruct((TOTAL, D), jnp.float32),
        mesh=mesh,
        compiler_params=pltpu.CompilerParams(
            use_tc_tiling_on_sc=True, needs_layout_passes=False
        ),
        scratch_types=[
            pltpu.VMEM((NCHUNK, C), jnp.int32),
            pltpu.VMEM((16,), jnp.int32),
            pltpu.VMEM((NB, C, D), jnp.float32),
            pltpu.SemaphoreType.DMA((NB,)),
            pltpu.SemaphoreType.DMA((NB,)),
        ],
    )
    def k(ids_hbm, offs_hbm, table_hbm, out_hbm, idx_v, offs_v, rows_v, gsem, osem):
        wid = lax.axis_index("s") * NC + lax.axis_index("c")
        rowbase = wid * NPW
        pltpu.sync_copy(ids_hbm.at[pl.ds(wid * NCHUNK, NCHUNK)], idx_v)
        pltpu.sync_copy(offs_hbm, offs_v)
        ov = offs_v[...]

        @pl.loop(0, NCHUNK)
        def _(j):
            for g in range(C // 16):
                sl = (j, pl.ds(g * 16, 16))
                idx_v[sl] = idx_v[sl] + ov

        def g_start(ch, b):
            @pl.loop(0, C // 16)
            def _(g):
                vec = idx_v[ch, pl.ds(g * 16, 16)]
                for lane in range(16):
                    r = vec[lane]
                    pltpu.async_copy(
                        table_hbm.at[
                            lax.shift_right_logical(r, 3),
                            lax.bitwise_and(r, 7),
                        ],
                        rows_v.at[b, g * 16 + lane],
                        gsem.at[b],
                    )

        def g_wait(ch, b):
            @pl.loop(0, C, unroll=8)
            def _(i):
                pltpu.make_async_copy(
                    table_hbm.at[0, 0], rows_v.at[b, i], gsem.at[b]
                ).wait()

        def o_start(ch, b):
            pltpu.async_copy(
                rows_v.at[b], out_hbm.at[pl.ds(rowbase + ch * C, C)], osem.at[b]
            )

        def o_wait(ch, b):
            pltpu.make_async_copy(
                rows_v.at[b], out_hbm.at[pl.ds(rowbase + ch * C, C)], osem.at[b]
            ).wait()

        for b in range(NB):
            g_start(b, b)

        @pl.loop(0, NCHUNK - NB, step=NB)
        def _(j):
            for b in range(NB):
                ch = j + b
                g_wait(ch, b)
                o_start(ch, b)
                o_wait(ch, b)
                g_start(ch + NB, b)

        for b in range(NB):
            ch = NCHUNK - NB + b
            g_wait(ch, b)
            o_start(ch, b)
        for b in range(NB):
            o_wait(NCHUNK - NB + b, b)

    return k


def kernel(hash_ids, table, offsets):
    ids2d = hash_ids.reshape(TOTAL // C, C)
    offs16 = jnp.concatenate([offsets, offsets]).astype(jnp.int32)
    table3 = table.reshape(V // 8, 8, D)
    out = _sc_gather(ids2d, offs16, table3)(ids2d, offs16, table3)
    return out.reshape(B, T, H, D)
